# Initial kernel scaffold; baseline (speedup 1.0000x reference)
import jax
import jax.numpy as jnp
from jax.experimental import pallas as pl


def kernel(x, edge_index, z, Wz1, bz1, Wz2, bz2, Wl1, bl1, Wr1, Wl2, bl2, Wr2, Wc0, bc0, Wc1, bc1, Wc2, bc2, Wc3, bc3):
    raise NotImplementedError("write your pallas kernel here")


# speedup vs baseline: 2.7505x; 2.7505x over previous
import functools

import jax
import jax.numpy as jnp
from jax import lax
from jax.experimental import pallas as pl
from jax.experimental.pallas import tpu as pltpu
from jax.experimental.pallas import tpu_sc as plsc

N = 10000
E = 160000
NSUB = 16
ROWS_A = 624
ROWS_B = N - ROWS_A * (NSUB - 1)
EDGES_PER_TILE = E // NSUB
BE = 80
NBLK = EDGES_PER_TILE // BE
NB = 2000
GRID = N // NB


def _elu(a):
    return jnp.where(a > 0, a, jnp.exp(a) - 1.0)


def _split_chunks(sid, do):
    @pl.when(sid < NSUB - 1)
    def _():
        for off, ln in ((0, 208), (208, 208), (416, 208)):
            do(off, ln)

    @pl.when(sid == NSUB - 1)
    def _():
        for off, ln in ((0, 208), (208, 208), (416, 224)):
            do(off, ln)


def _make_segsum(num_chunks, chunks_per_core):
    mesh = plsc.VectorSubcoreMesh(core_axis_name="c", subcore_axis_name="s", num_cores=2, num_subcores=16)
    out_type = [jax.ShapeDtypeStruct((N, 128), jnp.float32)
                for _ in range(num_chunks)]
    scratch = [
        pltpu.VMEM_SHARED((N, 128), jnp.float32),
        pltpu.VMEM((BE,), jnp.int32),
        pltpu.VMEM((BE,), jnp.int32),
        pltpu.VMEM((BE, 128), jnp.float32),
        pltpu.VMEM((224, 128), jnp.float32),
        pltpu.SemaphoreType.DMA,
    ]

    @functools.partial(pl.kernel, mesh=mesh, out_type=tuple(out_type),
                       scratch_types=tuple(scratch))
    def seg(*refs):
        tables = refs[:num_chunks]
        srcr, dstr, zeros128 = refs[num_chunks:num_chunks + 3]
        outs = refs[num_chunks + 3:2 * num_chunks + 3]
        acc, src_v, dst_v, rows_v, sbuf, sem = refs[2 * num_chunks + 3:]

        cid = lax.axis_index("c")
        sid = lax.axis_index("s")
        row0 = sid * ROWS_A
        ebase0 = sid * EDGES_PER_TILE

        for f in range(num_chunks):
            @pl.when(cid == f // chunks_per_core)
            def _(f=f):
                pltpu.sync_copy(zeros128, sbuf)

                def zinit(off, ln):
                    pltpu.sync_copy(sbuf.at[pl.ds(0, ln)],
                                    acc.at[pl.ds(row0 + off, ln)])

                _split_chunks(sid, zinit)
                plsc.subcore_barrier()

                def body(blk, carry, f=f):
                    base = ebase0 + blk * BE
                    pltpu.sync_copy(srcr.at[pl.ds(base, BE)], src_v)
                    pltpu.sync_copy(dstr.at[pl.ds(base, BE)], dst_v)
                    pltpu.async_copy(tables[f].at[src_v], rows_v, sem).wait()
                    pltpu.sync_copy(rows_v, acc.at[dst_v], add=True)
                    return carry

                lax.fori_loop(0, NBLK, body, 0)
                plsc.subcore_barrier()

                def wout(off, ln, f=f):
                    pltpu.sync_copy(acc.at[pl.ds(row0 + off, ln)],
                                    sbuf.at[pl.ds(0, ln)])
                    pltpu.sync_copy(sbuf.at[pl.ds(0, ln)],
                                    outs[f].at[pl.ds(row0 + off, ln)])

                _split_chunks(sid, wout)

    return seg


_segsum2 = _make_segsum(2, 1)
_segsum4 = _make_segsum(4, 2)

BC = 40
EDGES_PER_CTILE = E // 32
NCBLK = EDGES_PER_CTILE // BC

_counts_mesh = plsc.VectorSubcoreMesh(core_axis_name="c", subcore_axis_name="s", num_cores=2, num_subcores=16)


@functools.partial(
    pl.kernel, mesh=_counts_mesh,
    out_type=(jax.ShapeDtypeStruct((N, 128), jnp.float32),
              jax.ShapeDtypeStruct((N, 128), jnp.float32)),
    scratch_types=(
        pltpu.VMEM_SHARED((N, 128), jnp.float32),
        pltpu.VMEM((BC,), jnp.int32),
        pltpu.VMEM((BC, 128), jnp.float32),
        pltpu.VMEM((224, 128), jnp.float32),
    ))
def _counts(dstr, zeros128, ones128, out_a, out_b, cacc, dst_v, ones_v, cbuf):
    cid = lax.axis_index("c")
    sid = lax.axis_index("s")
    row0 = sid * ROWS_A
    pltpu.sync_copy(ones128, ones_v)
    pltpu.sync_copy(zeros128, cbuf)

    def zinit(off, ln):
        pltpu.sync_copy(cbuf.at[pl.ds(0, ln)], cacc.at[pl.ds(row0 + off, ln)])

    _split_chunks(sid, zinit)
    plsc.subcore_barrier()
    ebase0 = cid * (E // 2) + sid * EDGES_PER_CTILE

    def body(blk, carry):
        base = ebase0 + blk * BC
        pltpu.sync_copy(dstr.at[pl.ds(base, BC)], dst_v)
        pltpu.sync_copy(ones_v, cacc.at[dst_v], add=True)
        return carry

    lax.fori_loop(0, NCBLK, body, 0)
    plsc.subcore_barrier()

    for core, out in ((0, out_a), (1, out_b)):
        @pl.when(cid == core)
        def _(out=out):
            def cout(off, ln):
                pltpu.sync_copy(cacc.at[pl.ds(row0 + off, ln)],
                                cbuf.at[pl.ds(0, ln)])
                pltpu.sync_copy(cbuf.at[pl.ds(0, ln)],
                                out.at[pl.ds(row0 + off, ln)])

            _split_chunks(sid, cout)


def _dot(a, b):
    return jnp.dot(a, b, preferred_element_type=jnp.float32)


def _zhead_body(z_ref, wz1_ref, bz1_ref, wz2_ref, bz2_ref, wcz_ref, bc0_ref,
                out_ref):
    t = _elu(_dot(z_ref[...], wz1_ref[...]) + bz1_ref[...])
    t = _elu(_dot(t, wz2_ref[...]) + bz2_ref[...])
    out_ref[...] = _dot(t, wcz_ref[...]) + bc0_ref[...]


def _zhead(z, Wz1T, bz1, Wz2T, bz2, WczT, bc0):
    return pl.pallas_call(
        _zhead_body,
        out_shape=jax.ShapeDtypeStruct((400, 512), jnp.float32),
    )(z, Wz1T, bz1, Wz2T, bz2, WczT, bc0)


def _conv1_body(s0_ref, s1_ref, ca_ref, cb_ref, x0_ref, x1_ref, wl_ref,
                bl_ref, wr_ref, h0_ref, h1_ref, h2_ref, h3_ref):
    inv = 1.0 / jnp.clip(ca_ref[:, :1] + cb_ref[:, :1], 1.0, None)
    a = _dot(s0_ref[...] * inv, wl_ref[:128])
    a += _dot(s1_ref[...] * inv, wl_ref[128:])
    a += _dot(x0_ref[...], wr_ref[:128])
    a += _dot(x1_ref[...], wr_ref[128:])
    h = jnp.maximum(a + bl_ref[...], 0.0)
    h0_ref[...] = h[:, 0:128]
    h1_ref[...] = h[:, 128:256]
    h2_ref[...] = h[:, 256:384]
    h3_ref[...] = h[:, 384:512]


def _conv1(s0, s1, ca, cb, x0, x1, Wl1T, bl1, Wr1T):
    row = pl.BlockSpec((NB, 128), lambda i: (i, 0))
    full = lambda shape: pl.BlockSpec(shape, lambda i: (0, 0))
    cspec = pl.BlockSpec((NB, 128), lambda i: (i, 0))
    return pl.pallas_call(
        _conv1_body,
        grid=(GRID,),
        in_specs=[row, row, cspec, cspec,
                  row, row, full((256, 512)), full((1, 512)), full((256, 512))],
        out_specs=[row, row, row, row],
        out_shape=[jax.ShapeDtypeStruct((N, 128), jnp.float32)] * 4,
    )(s0, s1, ca, cb, x0, x1, Wl1T, bl1, Wr1T)


def _tail_body(t0_ref, t1_ref, t2_ref, t3_ref, h0_ref, h1_ref, h2_ref, h3_ref,
               ca_ref, cb_ref, zc_ref, wl_ref, bl_ref, wr_ref, wc0_ref,
               wc1_ref, bc1_ref, wc2_ref, bc2_ref, wc3_ref, bc3_ref, out_ref):
    inv = 1.0 / jnp.clip(ca_ref[:, :1] + cb_ref[:, :1], 1.0, None)
    t_refs = (t0_ref, t1_ref, t2_ref, t3_ref)
    h_refs = (h0_ref, h1_ref, h2_ref, h3_ref)
    a = bl_ref[...] + jnp.zeros((NB, 512), jnp.float32)
    for f in range(4):
        a += _dot(t_refs[f][...] * inv, wl_ref[pl.ds(128 * f, 128)])
        a += _dot(h_refs[f][...], wr_ref[pl.ds(128 * f, 128)])
    c = _dot(a, wc0_ref[...])
    c = (c.reshape(NB // 400, 400, 512) + zc_ref[...][None]).reshape(NB, 512)
    c = _elu(c)
    c = _elu(_dot(c, wc1_ref[...]) + bc1_ref[...])
    c = _elu(_dot(c, wc2_ref[...]) + bc2_ref[...])
    o = jax.nn.sigmoid(_elu(_dot(c, wc3_ref[...]) + bc3_ref[...]))
    out_ref[...] = o[:, 0:1]


def _tail(ts, hs, ca, cb, zc0, Wl2T, bl2, Wr2T, Wc0xT, Wc1T, bc1, Wc2T, bc2,
          Wc3p, bc3):
    row = pl.BlockSpec((NB, 128), lambda i: (i, 0))
    full = lambda shape: pl.BlockSpec(shape, lambda i: (0, 0))
    w = full((512, 512))
    b = full((1, 512))
    return pl.pallas_call(
        _tail_body,
        grid=(GRID,),
        in_specs=[row, row, row, row, row, row, row, row,
                  row, row,
                  full((400, 512)), w, b, w, w, w, b, w, b,
                  full((512, 128)), full((1, 128))],
        out_specs=pl.BlockSpec((NB, 1), lambda i: (i, 0)),
        out_shape=jax.ShapeDtypeStruct((N, 1), jnp.float32),
    )(*ts, *hs, ca, cb, zc0, Wl2T, bl2, Wr2T, Wc0xT, Wc1T, bc1, Wc2T, bc2,
      Wc3p, bc3)


def kernel(x, edge_index, z, Wz1, bz1, Wz2, bz2, Wl1, bl1, Wr1, Wl2, bl2,
           Wr2, Wc0, bc0, Wc1, bc1, Wc2, bc2, Wc3, bc3):
    f32 = jnp.float32
    src = edge_index[0]
    dst = edge_index[1]
    x0 = x[:, :128]
    x1 = x[:, 128:]
    zeros128 = jnp.zeros((224, 128), f32)
    ones128 = jnp.ones((BC, 128), f32)

    ca, cb = _counts(dst, zeros128, ones128)
    s0, s1 = _segsum2(x0, x1, src, dst, zeros128)

    zc0 = _zhead(z, Wz1.T, bz1.reshape(1, 256), Wz2.T,
                 bz2.reshape(1, 256), Wc0[:, 512:].T, bc0.reshape(1, 512))

    hs = _conv1(s0, s1, ca, cb, x0, x1, Wl1.T, bl1.reshape(1, 512), Wr1.T)

    ts = _segsum4(hs[0], hs[1], hs[2], hs[3], src, dst, zeros128)

    Wc3p = jnp.pad(Wc3.T, ((0, 0), (0, 127)))
    bc3p = jnp.pad(bc3.reshape(1, 1), ((0, 0), (0, 127)))
    out = _tail(ts, hs, ca, cb, zc0, Wl2.T, bl2.reshape(1, 512), Wr2.T,
                Wc0[:, :512].T, Wc1.T, bc1.reshape(1, 512), Wc2.T,
                bc2.reshape(1, 512), Wc3p, bc3p)
    return out


# speedup vs baseline: 4.1911x; 1.5238x over previous
import functools

import jax
import jax.numpy as jnp
from jax import lax
from jax.experimental import pallas as pl
from jax.experimental.pallas import tpu as pltpu
from jax.experimental.pallas import tpu_sc as plsc

N = 10000
E = 160000
NSUB = 16
ROWS_A = 624
ROWS_B = N - ROWS_A * (NSUB - 1)
EDGES_PER_TILE = E // NSUB
BE = 80
NBLK = EDGES_PER_TILE // BE
NB = 2000
GRID = N // NB


def _elu(a):
    return jnp.where(a > 0, a, jnp.exp(a) - 1.0)


def _split_chunks(sid, do):
    @pl.when(sid < NSUB - 1)
    def _():
        for off, ln in [(k * 80, 80) for k in range(7)] + [(560, 64)]:
            do(off, ln)

    @pl.when(sid == NSUB - 1)
    def _():
        for off, ln in [(k * 80, 80) for k in range(8)]:
            do(off, ln)


NBUF = 4
OUTER = NBLK // NBUF


def _make_segsum(num_chunks, chunks_per_core):
    mesh = plsc.VectorSubcoreMesh(core_axis_name="c", subcore_axis_name="s", num_cores=2, num_subcores=16)
    out_type = [jax.ShapeDtypeStruct((N, 128), jnp.float32)
                for _ in range(num_chunks)]
    scratch = [
        pltpu.VMEM_SHARED((N, 128), jnp.float32),
    ] + [pltpu.VMEM((BE, 128), jnp.float32)] * NBUF \
      + [pltpu.VMEM((BE,), jnp.int32)] * NBUF \
      + [pltpu.VMEM((BE,), jnp.int32)] * NBUF \
      + [pltpu.SemaphoreType.DMA] * NBUF

    @functools.partial(pl.kernel, mesh=mesh, out_type=tuple(out_type),
                       scratch_types=tuple(scratch))
    def seg(*refs):
        tables = refs[:num_chunks]
        srcr, dstr, zeros128 = refs[num_chunks:num_chunks + 3]
        outs = refs[num_chunks + 3:2 * num_chunks + 3]
        acc = refs[2 * num_chunks + 3]
        rest = refs[2 * num_chunks + 4:]
        ring = rest[:NBUF]
        srcv = rest[NBUF:2 * NBUF]
        dstv = rest[2 * NBUF:3 * NBUF]
        sems = rest[3 * NBUF:]

        cid = lax.axis_index("c")
        sid = lax.axis_index("s")
        row0 = sid * ROWS_A

        for f in range(num_chunks):
            @pl.when(cid == f // chunks_per_core)
            def _(f=f):
                pltpu.sync_copy(zeros128, ring[0])

                def zinit(off, ln):
                    pltpu.sync_copy(ring[0].at[pl.ds(0, ln)],
                                    acc.at[pl.ds(row0 + off, ln)])

                _split_chunks(sid, zinit)
                plsc.subcore_barrier()

                def outer(g, carry, f=f):
                    for b in range(NBUF):
                        j = g * NBUF + b
                        pltpu.sync_copy(srcr.at[sid, j, 0], srcv[b])
                        pltpu.sync_copy(dstr.at[sid, j, 0], dstv[b])
                        pltpu.async_copy(tables[f].at[srcv[b]],
                                         ring[b], sems[b])
                    for b in range(NBUF):
                        pltpu.make_async_copy(tables[f].at[srcv[b]],
                                              ring[b], sems[b]).wait()
                        pltpu.sync_copy(ring[b], acc.at[dstv[b]], add=True)
                    return carry

                lax.fori_loop(0, OUTER, outer, 0)
                for j in range(OUTER * NBUF, NBLK):
                    pltpu.sync_copy(srcr.at[sid, j, 0], srcv[0])
                    pltpu.sync_copy(dstr.at[sid, j, 0], dstv[0])
                    pltpu.async_copy(tables[f].at[srcv[0]],
                                     ring[0], sems[0]).wait()
                    pltpu.sync_copy(ring[0], acc.at[dstv[0]], add=True)
                plsc.subcore_barrier()

                def wout(off, ln, f=f):
                    pltpu.sync_copy(acc.at[pl.ds(row0 + off, ln)],
                                    ring[0].at[pl.ds(0, ln)])
                    pltpu.sync_copy(ring[0].at[pl.ds(0, ln)],
                                    outs[f].at[pl.ds(row0 + off, ln)])

                _split_chunks(sid, wout)

    return seg


_segsum2 = _make_segsum(2, 1)
_segsum4 = _make_segsum(4, 2)

BC = 40
EDGES_PER_CTILE = E // 32
NCBLK = EDGES_PER_CTILE // BC

_counts_mesh = plsc.VectorSubcoreMesh(core_axis_name="c", subcore_axis_name="s", num_cores=2, num_subcores=16)


@functools.partial(
    pl.kernel, mesh=_counts_mesh,
    out_type=(jax.ShapeDtypeStruct((N, 128), jnp.float32),
              jax.ShapeDtypeStruct((N, 128), jnp.float32)),
    scratch_types=(
        pltpu.VMEM_SHARED((N, 128), jnp.float32),
        pltpu.VMEM((NCBLK, 1, BC), jnp.int32),
        pltpu.VMEM((BC, 128), jnp.float32),
        pltpu.VMEM((80, 128), jnp.float32),
    ))
def _counts(dstr, zeros128, ones128, out_a, out_b, cacc, dst2d, ones_v, cbuf):
    cid = lax.axis_index("c")
    sid = lax.axis_index("s")
    row0 = sid * ROWS_A
    pltpu.sync_copy(ones128, ones_v)
    pltpu.sync_copy(zeros128, cbuf)
    pltpu.sync_copy(dstr.at[cid, sid], dst2d)

    def zinit(off, ln):
        pltpu.sync_copy(cbuf.at[pl.ds(0, ln)], cacc.at[pl.ds(row0 + off, ln)])

    _split_chunks(sid, zinit)
    plsc.subcore_barrier()

    def body(blk, carry):
        pltpu.sync_copy(ones_v, cacc.at[dst2d.at[blk, 0]], add=True)
        return carry

    lax.fori_loop(0, NCBLK, body, 0)
    plsc.subcore_barrier()

    for core, out in ((0, out_a), (1, out_b)):
        @pl.when(cid == core)
        def _(out=out):
            def cout(off, ln):
                pltpu.sync_copy(cacc.at[pl.ds(row0 + off, ln)],
                                cbuf.at[pl.ds(0, ln)])
                pltpu.sync_copy(cbuf.at[pl.ds(0, ln)],
                                out.at[pl.ds(row0 + off, ln)])

            _split_chunks(sid, cout)


def _dot(a, b):
    return jnp.dot(a, b, preferred_element_type=jnp.float32)


def _zhead_body(z_ref, wz1_ref, bz1_ref, wz2_ref, bz2_ref, wcz_ref, bc0_ref,
                out_ref):
    t = _elu(_dot(z_ref[...], wz1_ref[...]) + bz1_ref[...])
    t = _elu(_dot(t, wz2_ref[...]) + bz2_ref[...])
    out_ref[...] = _dot(t, wcz_ref[...]) + bc0_ref[...]


def _zhead(z, Wz1T, bz1, Wz2T, bz2, WczT, bc0):
    return pl.pallas_call(
        _zhead_body,
        out_shape=jax.ShapeDtypeStruct((400, 512), jnp.float32),
    )(z, Wz1T, bz1, Wz2T, bz2, WczT, bc0)


def _conv1_body(s0_ref, s1_ref, ca_ref, cb_ref, x0_ref, x1_ref, wl_ref,
                bl_ref, wr_ref, h0_ref, h1_ref, h2_ref, h3_ref):
    inv = 1.0 / jnp.clip(ca_ref[:, :1] + cb_ref[:, :1], 1.0, None)
    a = _dot(s0_ref[...] * inv, wl_ref[:128])
    a += _dot(s1_ref[...] * inv, wl_ref[128:])
    a += _dot(x0_ref[...], wr_ref[:128])
    a += _dot(x1_ref[...], wr_ref[128:])
    h = jnp.maximum(a + bl_ref[...], 0.0)
    h0_ref[...] = h[:, 0:128]
    h1_ref[...] = h[:, 128:256]
    h2_ref[...] = h[:, 256:384]
    h3_ref[...] = h[:, 384:512]


def _conv1(s0, s1, ca, cb, x0, x1, Wl1T, bl1, Wr1T):
    row = pl.BlockSpec((NB, 128), lambda i: (i, 0))
    full = lambda shape: pl.BlockSpec(shape, lambda i: (0, 0))
    cspec = pl.BlockSpec((NB, 128), lambda i: (i, 0))
    return pl.pallas_call(
        _conv1_body,
        grid=(GRID,),
        in_specs=[row, row, cspec, cspec,
                  row, row, full((256, 512)), full((1, 512)), full((256, 512))],
        out_specs=[row, row, row, row],
        out_shape=[jax.ShapeDtypeStruct((N, 128), jnp.float32)] * 4,
    )(s0, s1, ca, cb, x0, x1, Wl1T, bl1, Wr1T)


def _tail_body(t0_ref, t1_ref, t2_ref, t3_ref, h0_ref, h1_ref, h2_ref, h3_ref,
               ca_ref, cb_ref, zc_ref, wl_ref, bl_ref, wr_ref, wc0_ref,
               wc1_ref, bc1_ref, wc2_ref, bc2_ref, wc3_ref, bc3_ref, out_ref):
    inv = 1.0 / jnp.clip(ca_ref[:, :1] + cb_ref[:, :1], 1.0, None)
    t_refs = (t0_ref, t1_ref, t2_ref, t3_ref)
    h_refs = (h0_ref, h1_ref, h2_ref, h3_ref)
    a = bl_ref[...] + jnp.zeros((NB, 512), jnp.float32)
    for f in range(4):
        a += _dot(t_refs[f][...] * inv, wl_ref[pl.ds(128 * f, 128)])
        a += _dot(h_refs[f][...], wr_ref[pl.ds(128 * f, 128)])
    c = _dot(a, wc0_ref[...])
    c = (c.reshape(NB // 400, 400, 512) + zc_ref[...][None]).reshape(NB, 512)
    c = _elu(c)
    c = _elu(_dot(c, wc1_ref[...]) + bc1_ref[...])
    c = _elu(_dot(c, wc2_ref[...]) + bc2_ref[...])
    o = jax.nn.sigmoid(_elu(_dot(c, wc3_ref[...]) + bc3_ref[...]))
    out_ref[...] = o[:, 0:1]


def _tail(ts, hs, ca, cb, zc0, Wl2T, bl2, Wr2T, Wc0xT, Wc1T, bc1, Wc2T, bc2,
          Wc3p, bc3):
    row = pl.BlockSpec((NB, 128), lambda i: (i, 0))
    full = lambda shape: pl.BlockSpec(shape, lambda i: (0, 0))
    w = full((512, 512))
    b = full((1, 512))
    return pl.pallas_call(
        _tail_body,
        grid=(GRID,),
        in_specs=[row, row, row, row, row, row, row, row,
                  row, row,
                  full((400, 512)), w, b, w, w, w, b, w, b,
                  full((512, 128)), full((1, 128))],
        out_specs=pl.BlockSpec((NB, 1), lambda i: (i, 0)),
        out_shape=jax.ShapeDtypeStruct((N, 1), jnp.float32),
    )(*ts, *hs, ca, cb, zc0, Wl2T, bl2, Wr2T, Wc0xT, Wc1T, bc1, Wc2T, bc2,
      Wc3p, bc3)


def kernel(x, edge_index, z, Wz1, bz1, Wz2, bz2, Wl1, bl1, Wr1, Wl2, bl2,
           Wr2, Wc0, bc0, Wc1, bc1, Wc2, bc2, Wc3, bc3):
    f32 = jnp.float32
    src = edge_index[0].reshape(NSUB, NBLK, 1, BE)
    dst = edge_index[1].reshape(NSUB, NBLK, 1, BE)
    dstc = edge_index[1].reshape(2, NSUB, NCBLK, 1, BC)
    x0 = x[:, :128]
    x1 = x[:, 128:]
    zeros128 = jnp.zeros((80, 128), f32)
    ones128 = jnp.ones((BC, 128), f32)

    ca, cb = _counts(dstc, zeros128, ones128)
    s0, s1 = _segsum2(x0, x1, src, dst, zeros128)

    zc0 = _zhead(z, Wz1.T, bz1.reshape(1, 256), Wz2.T,
                 bz2.reshape(1, 256), Wc0[:, 512:].T, bc0.reshape(1, 512))

    hs = _conv1(s0, s1, ca, cb, x0, x1, Wl1.T, bl1.reshape(1, 512), Wr1.T)

    ts = _segsum4(hs[0], hs[1], hs[2], hs[3], src, dst, zeros128)

    Wc3p = jnp.pad(Wc3.T, ((0, 0), (0, 127)))
    bc3p = jnp.pad(bc3.reshape(1, 1), ((0, 0), (0, 127)))
    out = _tail(ts, hs, ca, cb, zc0, Wl2.T, bl2.reshape(1, 512), Wr2.T,
                Wc0[:, :512].T, Wc1.T, bc1.reshape(1, 512), Wc2.T,
                bc2.reshape(1, 512), Wc3p, bc3p)
    return out


# speedup vs baseline: 4.2123x; 1.0051x over previous
import functools

import jax
import jax.numpy as jnp
from jax import lax
from jax.experimental import pallas as pl
from jax.experimental.pallas import tpu as pltpu
from jax.experimental.pallas import tpu_sc as plsc

N = 10000
E = 160000
NSUB = 16
ROWS_A = 624
ROWS_B = N - ROWS_A * (NSUB - 1)
EDGES_PER_TILE = E // NSUB
BE = 80
NBLK = EDGES_PER_TILE // BE
NB = 2000
GRID = N // NB


def _elu(a):
    return jnp.where(a > 0, a, jnp.exp(a) - 1.0)


def _split_chunks(sid, do):
    @pl.when(sid < NSUB - 1)
    def _():
        for off, ln in [(k * 80, 80) for k in range(7)] + [(560, 64)]:
            do(off, ln)

    @pl.when(sid == NSUB - 1)
    def _():
        for off, ln in [(k * 80, 80) for k in range(8)]:
            do(off, ln)


NBUF = 4
OUTER = NBLK // NBUF


def _make_segsum(num_chunks, chunks_per_core):
    mesh = plsc.VectorSubcoreMesh(core_axis_name="c", subcore_axis_name="s", num_cores=2, num_subcores=16)
    out_type = [jax.ShapeDtypeStruct((N, 128), jnp.float32)
                for _ in range(num_chunks)]
    scratch = [
        pltpu.VMEM_SHARED((N, 128), jnp.float32),
    ] + [pltpu.VMEM((BE, 128), jnp.float32)] * NBUF \
      + [pltpu.VMEM((BE,), jnp.int32)] * NBUF \
      + [pltpu.VMEM((BE,), jnp.int32)] * NBUF \
      + [pltpu.SemaphoreType.DMA] * NBUF

    @functools.partial(pl.kernel, mesh=mesh, out_type=tuple(out_type),
                       scratch_types=tuple(scratch))
    def seg(*refs):
        tables = refs[:num_chunks]
        srcr, dstr, zeros128 = refs[num_chunks:num_chunks + 3]
        outs = refs[num_chunks + 3:2 * num_chunks + 3]
        acc = refs[2 * num_chunks + 3]
        rest = refs[2 * num_chunks + 4:]
        ring = rest[:NBUF]
        srcv = rest[NBUF:2 * NBUF]
        dstv = rest[2 * NBUF:3 * NBUF]
        sems = rest[3 * NBUF:]

        cid = lax.axis_index("c")
        sid = lax.axis_index("s")
        row0 = sid * ROWS_A

        for f in range(num_chunks):
            @pl.when(cid == f // chunks_per_core)
            def _(f=f):
                pltpu.sync_copy(zeros128, ring[0])

                def zinit(off, ln):
                    pltpu.sync_copy(ring[0].at[pl.ds(0, ln)],
                                    acc.at[pl.ds(row0 + off, ln)])

                _split_chunks(sid, zinit)
                plsc.subcore_barrier()

                for b in range(NBUF):
                    pltpu.sync_copy(srcr.at[sid, b, 0], srcv[b])
                    pltpu.sync_copy(dstr.at[sid, b, 0], dstv[b])
                    pltpu.async_copy(tables[f].at[srcv[b]],
                                     ring[b], sems[b])

                def outer(g, carry, f=f):
                    for b in range(NBUF):
                        j = g * NBUF + b
                        pltpu.make_async_copy(tables[f].at[srcv[b]],
                                              ring[b], sems[b]).wait()
                        pltpu.sync_copy(ring[b], acc.at[dstv[b]], add=True)

                        @pl.when(j + NBUF < NBLK - 1)
                        def _(b=b, j=j):
                            pltpu.sync_copy(srcr.at[sid, j + NBUF, 0],
                                            srcv[b])
                            pltpu.sync_copy(dstr.at[sid, j + NBUF, 0],
                                            dstv[b])
                            pltpu.async_copy(tables[f].at[srcv[b]],
                                             ring[b], sems[b])
                    return carry

                lax.fori_loop(0, OUTER, outer, 0)
                for j in range(OUTER * NBUF, NBLK):
                    pltpu.sync_copy(srcr.at[sid, j, 0], srcv[0])
                    pltpu.sync_copy(dstr.at[sid, j, 0], dstv[0])
                    pltpu.async_copy(tables[f].at[srcv[0]],
                                     ring[0], sems[0]).wait()
                    pltpu.sync_copy(ring[0], acc.at[dstv[0]], add=True)
                plsc.subcore_barrier()

                def wout(off, ln, f=f):
                    pltpu.sync_copy(acc.at[pl.ds(row0 + off, ln)],
                                    ring[0].at[pl.ds(0, ln)])
                    pltpu.sync_copy(ring[0].at[pl.ds(0, ln)],
                                    outs[f].at[pl.ds(row0 + off, ln)])

                _split_chunks(sid, wout)

    return seg


_segsum2 = _make_segsum(2, 1)
_segsum4 = _make_segsum(4, 2)

BC = 40
EDGES_PER_CTILE = E // 32
NCBLK = EDGES_PER_CTILE // BC

_counts_mesh = plsc.VectorSubcoreMesh(core_axis_name="c", subcore_axis_name="s", num_cores=2, num_subcores=16)


@functools.partial(
    pl.kernel, mesh=_counts_mesh,
    out_type=(jax.ShapeDtypeStruct((N, 128), jnp.float32),
              jax.ShapeDtypeStruct((N, 128), jnp.float32)),
    scratch_types=(
        pltpu.VMEM_SHARED((N, 128), jnp.float32),
        pltpu.VMEM((NCBLK, 1, BC), jnp.int32),
        pltpu.VMEM((BC, 128), jnp.float32),
        pltpu.VMEM((80, 128), jnp.float32),
    ))
def _counts(dstr, zeros128, ones128, out_a, out_b, cacc, dst2d, ones_v, cbuf):
    cid = lax.axis_index("c")
    sid = lax.axis_index("s")
    row0 = sid * ROWS_A
    pltpu.sync_copy(ones128, ones_v)
    pltpu.sync_copy(zeros128, cbuf)
    pltpu.sync_copy(dstr.at[cid, sid], dst2d)

    def zinit(off, ln):
        pltpu.sync_copy(cbuf.at[pl.ds(0, ln)], cacc.at[pl.ds(row0 + off, ln)])

    _split_chunks(sid, zinit)
    plsc.subcore_barrier()

    def body(blk, carry):
        pltpu.sync_copy(ones_v, cacc.at[dst2d.at[blk, 0]], add=True)
        return carry

    lax.fori_loop(0, NCBLK, body, 0)
    plsc.subcore_barrier()

    for core, out in ((0, out_a), (1, out_b)):
        @pl.when(cid == core)
        def _(out=out):
            def cout(off, ln):
                pltpu.sync_copy(cacc.at[pl.ds(row0 + off, ln)],
                                cbuf.at[pl.ds(0, ln)])
                pltpu.sync_copy(cbuf.at[pl.ds(0, ln)],
                                out.at[pl.ds(row0 + off, ln)])

            _split_chunks(sid, cout)


def _dot(a, b):
    return jnp.dot(a, b, preferred_element_type=jnp.float32)


def _zhead_body(z_ref, wz1_ref, bz1_ref, wz2_ref, bz2_ref, wcz_ref, bc0_ref,
                out_ref):
    t = _elu(_dot(z_ref[...], wz1_ref[...]) + bz1_ref[...])
    t = _elu(_dot(t, wz2_ref[...]) + bz2_ref[...])
    out_ref[...] = _dot(t, wcz_ref[...]) + bc0_ref[...]


def _zhead(z, Wz1T, bz1, Wz2T, bz2, WczT, bc0):
    return pl.pallas_call(
        _zhead_body,
        out_shape=jax.ShapeDtypeStruct((400, 512), jnp.float32),
    )(z, Wz1T, bz1, Wz2T, bz2, WczT, bc0)


def _conv1_body(s0_ref, s1_ref, ca_ref, cb_ref, x0_ref, x1_ref, wl_ref,
                bl_ref, wr_ref, h0_ref, h1_ref, h2_ref, h3_ref):
    inv = 1.0 / jnp.clip(ca_ref[:, :1] + cb_ref[:, :1], 1.0, None)
    a = _dot(s0_ref[...] * inv, wl_ref[:128])
    a += _dot(s1_ref[...] * inv, wl_ref[128:])
    a += _dot(x0_ref[...], wr_ref[:128])
    a += _dot(x1_ref[...], wr_ref[128:])
    h = jnp.maximum(a + bl_ref[...], 0.0)
    h0_ref[...] = h[:, 0:128]
    h1_ref[...] = h[:, 128:256]
    h2_ref[...] = h[:, 256:384]
    h3_ref[...] = h[:, 384:512]


def _conv1(s0, s1, ca, cb, x0, x1, Wl1T, bl1, Wr1T):
    row = pl.BlockSpec((NB, 128), lambda i: (i, 0))
    full = lambda shape: pl.BlockSpec(shape, lambda i: (0, 0))
    cspec = pl.BlockSpec((NB, 128), lambda i: (i, 0))
    return pl.pallas_call(
        _conv1_body,
        grid=(GRID,),
        in_specs=[row, row, cspec, cspec,
                  row, row, full((256, 512)), full((1, 512)), full((256, 512))],
        out_specs=[row, row, row, row],
        out_shape=[jax.ShapeDtypeStruct((N, 128), jnp.float32)] * 4,
    )(s0, s1, ca, cb, x0, x1, Wl1T, bl1, Wr1T)


def _tail_body(t0_ref, t1_ref, t2_ref, t3_ref, h0_ref, h1_ref, h2_ref, h3_ref,
               ca_ref, cb_ref, zc_ref, wl_ref, bl_ref, wr_ref, wc0_ref,
               wc1_ref, bc1_ref, wc2_ref, bc2_ref, wc3_ref, bc3_ref, out_ref):
    inv = 1.0 / jnp.clip(ca_ref[:, :1] + cb_ref[:, :1], 1.0, None)
    t_refs = (t0_ref, t1_ref, t2_ref, t3_ref)
    h_refs = (h0_ref, h1_ref, h2_ref, h3_ref)
    a = bl_ref[...] + jnp.zeros((NB, 512), jnp.float32)
    for f in range(4):
        a += _dot(t_refs[f][...] * inv, wl_ref[pl.ds(128 * f, 128)])
        a += _dot(h_refs[f][...], wr_ref[pl.ds(128 * f, 128)])
    c = _dot(a, wc0_ref[...])
    c = (c.reshape(NB // 400, 400, 512) + zc_ref[...][None]).reshape(NB, 512)
    c = _elu(c)
    c = _elu(_dot(c, wc1_ref[...]) + bc1_ref[...])
    c = _elu(_dot(c, wc2_ref[...]) + bc2_ref[...])
    o = jax.nn.sigmoid(_elu(_dot(c, wc3_ref[...]) + bc3_ref[...]))
    out_ref[...] = o[:, 0:1]


def _tail(ts, hs, ca, cb, zc0, Wl2T, bl2, Wr2T, Wc0xT, Wc1T, bc1, Wc2T, bc2,
          Wc3p, bc3):
    row = pl.BlockSpec((NB, 128), lambda i: (i, 0))
    full = lambda shape: pl.BlockSpec(shape, lambda i: (0, 0))
    w = full((512, 512))
    b = full((1, 512))
    return pl.pallas_call(
        _tail_body,
        grid=(GRID,),
        in_specs=[row, row, row, row, row, row, row, row,
                  row, row,
                  full((400, 512)), w, b, w, w, w, b, w, b,
                  full((512, 128)), full((1, 128))],
        out_specs=pl.BlockSpec((NB, 1), lambda i: (i, 0)),
        out_shape=jax.ShapeDtypeStruct((N, 1), jnp.float32),
    )(*ts, *hs, ca, cb, zc0, Wl2T, bl2, Wr2T, Wc0xT, Wc1T, bc1, Wc2T, bc2,
      Wc3p, bc3)


def kernel(x, edge_index, z, Wz1, bz1, Wz2, bz2, Wl1, bl1, Wr1, Wl2, bl2,
           Wr2, Wc0, bc0, Wc1, bc1, Wc2, bc2, Wc3, bc3):
    f32 = jnp.float32
    src = edge_index[0].reshape(NSUB, NBLK, 1, BE)
    dst = edge_index[1].reshape(NSUB, NBLK, 1, BE)
    dstc = edge_index[1].reshape(2, NSUB, NCBLK, 1, BC)
    x0 = x[:, :128]
    x1 = x[:, 128:]
    zeros128 = jnp.zeros((80, 128), f32)
    ones128 = jnp.ones((BC, 128), f32)

    ca, cb = _counts(dstc, zeros128, ones128)
    s0, s1 = _segsum2(x0, x1, src, dst, zeros128)

    zc0 = _zhead(z, Wz1.T, bz1.reshape(1, 256), Wz2.T,
                 bz2.reshape(1, 256), Wc0[:, 512:].T, bc0.reshape(1, 512))

    hs = _conv1(s0, s1, ca, cb, x0, x1, Wl1.T, bl1.reshape(1, 512), Wr1.T)

    ts = _segsum4(hs[0], hs[1], hs[2], hs[3], src, dst, zeros128)

    Wc3p = jnp.pad(Wc3.T, ((0, 0), (0, 127)))
    bc3p = jnp.pad(bc3.reshape(1, 1), ((0, 0), (0, 127)))
    out = _tail(ts, hs, ca, cb, zc0, Wl2.T, bl2.reshape(1, 512), Wr2.T,
                Wc0[:, :512].T, Wc1.T, bc1.reshape(1, 512), Wc2.T,
                bc2.reshape(1, 512), Wc3p, bc3p)
    return out


# speedup vs baseline: 4.5323x; 1.0759x over previous
import functools

import jax
import jax.numpy as jnp
from jax import lax
from jax.experimental import pallas as pl
from jax.experimental.pallas import tpu as pltpu
from jax.experimental.pallas import tpu_sc as plsc

N = 10000
E = 160000
NSUB = 16
ROWS_A = 624
ROWS_B = N - ROWS_A * (NSUB - 1)
EDGES_PER_TILE = E // NSUB
BE = 80
NBLK = EDGES_PER_TILE // BE
NB = 2000
GRID = N // NB


def _elu(a):
    return jnp.where(a > 0, a, jnp.exp(a) - 1.0)


def _split_chunks(sid, do):
    @pl.when(sid < NSUB - 1)
    def _():
        for off, ln in [(k * 80, 80) for k in range(7)] + [(560, 64)]:
            do(off, ln)

    @pl.when(sid == NSUB - 1)
    def _():
        for off, ln in [(k * 80, 80) for k in range(8)]:
            do(off, ln)


NBUF = 4
OUTER = NBLK // NBUF


def _make_segsum(num_chunks, chunks_per_core):
    mesh = plsc.VectorSubcoreMesh(core_axis_name="c", subcore_axis_name="s", num_cores=2, num_subcores=16)
    out_type = [jax.ShapeDtypeStruct((N, 128), jnp.float32)
                for _ in range(num_chunks)]
    scratch = [
        pltpu.VMEM_SHARED((N, 128), jnp.float32),
    ] + [pltpu.VMEM((BE, 128), jnp.float32)] * NBUF \
      + [pltpu.VMEM((BE,), jnp.int32)] * NBUF \
      + [pltpu.VMEM((BE,), jnp.int32)] * NBUF \
      + [pltpu.SemaphoreType.DMA] * (2 * NBUF)

    @functools.partial(pl.kernel, mesh=mesh, out_type=tuple(out_type),
                       scratch_types=tuple(scratch))
    def seg(*refs):
        tables = refs[:num_chunks]
        srcr, dstr, zeros128 = refs[num_chunks:num_chunks + 3]
        outs = refs[num_chunks + 3:2 * num_chunks + 3]
        acc = refs[2 * num_chunks + 3]
        rest = refs[2 * num_chunks + 4:]
        ring = rest[:NBUF]
        srcv = rest[NBUF:2 * NBUF]
        dstv = rest[2 * NBUF:3 * NBUF]
        sems = rest[3 * NBUF:4 * NBUF]
        ssems = rest[4 * NBUF:]

        cid = lax.axis_index("c")
        sid = lax.axis_index("s")
        row0 = sid * ROWS_A

        for f in range(num_chunks):
            @pl.when(cid == f // chunks_per_core)
            def _(f=f):
                pltpu.sync_copy(zeros128, ring[0])

                def zinit(off, ln):
                    pltpu.sync_copy(ring[0].at[pl.ds(0, ln)],
                                    acc.at[pl.ds(row0 + off, ln)])

                _split_chunks(sid, zinit)
                plsc.subcore_barrier()

                for b in range(NBUF):
                    pltpu.sync_copy(srcr.at[sid, b, 0], srcv[b])
                    pltpu.sync_copy(dstr.at[sid, b, 0], dstv[b])
                    pltpu.async_copy(tables[f].at[srcv[b]],
                                     ring[b], sems[b])

                def outer(g, carry, f=f):
                    for b in range(NBUF):
                        pltpu.make_async_copy(tables[f].at[srcv[b]],
                                              ring[b], sems[b]).wait()
                        pltpu.async_copy(ring[b], acc.at[dstv[b]],
                                        ssems[b], add=True)
                    for b in range(NBUF):
                        j = g * NBUF + b
                        pltpu.make_async_copy(ring[b], acc.at[dstv[b]],
                                              ssems[b]).wait()

                        @pl.when(j + NBUF < NBLK - 1)
                        def _(b=b, j=j):
                            pltpu.sync_copy(srcr.at[sid, j + NBUF, 0],
                                            srcv[b])
                            pltpu.sync_copy(dstr.at[sid, j + NBUF, 0],
                                            dstv[b])
                            pltpu.async_copy(tables[f].at[srcv[b]],
                                             ring[b], sems[b])
                    return carry

                lax.fori_loop(0, OUTER, outer, 0)
                for j in range(OUTER * NBUF, NBLK):
                    pltpu.sync_copy(srcr.at[sid, j, 0], srcv[0])
                    pltpu.sync_copy(dstr.at[sid, j, 0], dstv[0])
                    pltpu.async_copy(tables[f].at[srcv[0]],
                                     ring[0], sems[0]).wait()
                    pltpu.sync_copy(ring[0], acc.at[dstv[0]], add=True)
                plsc.subcore_barrier()

                def wout(off, ln, f=f):
                    pltpu.sync_copy(acc.at[pl.ds(row0 + off, ln)],
                                    ring[0].at[pl.ds(0, ln)])
                    pltpu.sync_copy(ring[0].at[pl.ds(0, ln)],
                                    outs[f].at[pl.ds(row0 + off, ln)])

                _split_chunks(sid, wout)

    return seg


_segsum2 = _make_segsum(2, 1)
_segsum4 = _make_segsum(4, 2)

BC = 40
EDGES_PER_CTILE = E // 32
NCBLK = EDGES_PER_CTILE // BC

_counts_mesh = plsc.VectorSubcoreMesh(core_axis_name="c", subcore_axis_name="s", num_cores=2, num_subcores=16)


@functools.partial(
    pl.kernel, mesh=_counts_mesh,
    out_type=(jax.ShapeDtypeStruct((N, 128), jnp.float32),
              jax.ShapeDtypeStruct((N, 128), jnp.float32)),
    scratch_types=(
        pltpu.VMEM_SHARED((N, 128), jnp.float32),
        pltpu.VMEM((NCBLK, 1, BC), jnp.int32),
        pltpu.VMEM((BC, 128), jnp.float32),
        pltpu.VMEM((80, 128), jnp.float32),
    ))
def _counts(dstr, zeros128, ones128, out_a, out_b, cacc, dst2d, ones_v, cbuf):
    cid = lax.axis_index("c")
    sid = lax.axis_index("s")
    row0 = sid * ROWS_A
    pltpu.sync_copy(ones128, ones_v)
    pltpu.sync_copy(zeros128, cbuf)
    pltpu.sync_copy(dstr.at[cid, sid], dst2d)

    def zinit(off, ln):
        pltpu.sync_copy(cbuf.at[pl.ds(0, ln)], cacc.at[pl.ds(row0 + off, ln)])

    _split_chunks(sid, zinit)
    plsc.subcore_barrier()

    def body(blk, carry):
        pltpu.sync_copy(ones_v, cacc.at[dst2d.at[blk, 0]], add=True)
        return carry

    lax.fori_loop(0, NCBLK, body, 0)
    plsc.subcore_barrier()

    for core, out in ((0, out_a), (1, out_b)):
        @pl.when(cid == core)
        def _(out=out):
            def cout(off, ln):
                pltpu.sync_copy(cacc.at[pl.ds(row0 + off, ln)],
                                cbuf.at[pl.ds(0, ln)])
                pltpu.sync_copy(cbuf.at[pl.ds(0, ln)],
                                out.at[pl.ds(row0 + off, ln)])

            _split_chunks(sid, cout)


def _dot(a, b):
    return jnp.dot(a, b, preferred_element_type=jnp.float32)


def _zhead_body(z_ref, wz1_ref, bz1_ref, wz2_ref, bz2_ref, wcz_ref, bc0_ref,
                out_ref):
    t = _elu(_dot(z_ref[...], wz1_ref[...]) + bz1_ref[...])
    t = _elu(_dot(t, wz2_ref[...]) + bz2_ref[...])
    out_ref[...] = _dot(t, wcz_ref[...]) + bc0_ref[...]


def _zhead(z, Wz1T, bz1, Wz2T, bz2, WczT, bc0):
    return pl.pallas_call(
        _zhead_body,
        out_shape=jax.ShapeDtypeStruct((400, 512), jnp.float32),
    )(z, Wz1T, bz1, Wz2T, bz2, WczT, bc0)


def _conv1_body(s0_ref, s1_ref, ca_ref, cb_ref, x0_ref, x1_ref, wl_ref,
                bl_ref, wr_ref, h0_ref, h1_ref, h2_ref, h3_ref):
    inv = 1.0 / jnp.clip(ca_ref[:, :1] + cb_ref[:, :1], 1.0, None)
    a = _dot(s0_ref[...] * inv, wl_ref[:128])
    a += _dot(s1_ref[...] * inv, wl_ref[128:])
    a += _dot(x0_ref[...], wr_ref[:128])
    a += _dot(x1_ref[...], wr_ref[128:])
    h = jnp.maximum(a + bl_ref[...], 0.0)
    h0_ref[...] = h[:, 0:128]
    h1_ref[...] = h[:, 128:256]
    h2_ref[...] = h[:, 256:384]
    h3_ref[...] = h[:, 384:512]


def _conv1(s0, s1, ca, cb, x0, x1, Wl1T, bl1, Wr1T):
    row = pl.BlockSpec((NB, 128), lambda i: (i, 0))
    full = lambda shape: pl.BlockSpec(shape, lambda i: (0, 0))
    cspec = pl.BlockSpec((NB, 128), lambda i: (i, 0))
    return pl.pallas_call(
        _conv1_body,
        grid=(GRID,),
        in_specs=[row, row, cspec, cspec,
                  row, row, full((256, 512)), full((1, 512)), full((256, 512))],
        out_specs=[row, row, row, row],
        out_shape=[jax.ShapeDtypeStruct((N, 128), jnp.float32)] * 4,
    )(s0, s1, ca, cb, x0, x1, Wl1T, bl1, Wr1T)


def _tail_body(t0_ref, t1_ref, t2_ref, t3_ref, h0_ref, h1_ref, h2_ref, h3_ref,
               ca_ref, cb_ref, zc_ref, wl_ref, bl_ref, wr_ref, wc0_ref,
               wc1_ref, bc1_ref, wc2_ref, bc2_ref, wc3_ref, bc3_ref, out_ref):
    inv = 1.0 / jnp.clip(ca_ref[:, :1] + cb_ref[:, :1], 1.0, None)
    t_refs = (t0_ref, t1_ref, t2_ref, t3_ref)
    h_refs = (h0_ref, h1_ref, h2_ref, h3_ref)
    a = bl_ref[...] + jnp.zeros((NB, 512), jnp.float32)
    for f in range(4):
        a += _dot(t_refs[f][...] * inv, wl_ref[pl.ds(128 * f, 128)])
        a += _dot(h_refs[f][...], wr_ref[pl.ds(128 * f, 128)])
    c = _dot(a, wc0_ref[...])
    c = (c.reshape(NB // 400, 400, 512) + zc_ref[...][None]).reshape(NB, 512)
    c = _elu(c)
    c = _elu(_dot(c, wc1_ref[...]) + bc1_ref[...])
    c = _elu(_dot(c, wc2_ref[...]) + bc2_ref[...])
    o = jax.nn.sigmoid(_elu(_dot(c, wc3_ref[...]) + bc3_ref[...]))
    out_ref[...] = o[:, 0:1]


def _tail(ts, hs, ca, cb, zc0, Wl2T, bl2, Wr2T, Wc0xT, Wc1T, bc1, Wc2T, bc2,
          Wc3p, bc3):
    row = pl.BlockSpec((NB, 128), lambda i: (i, 0))
    full = lambda shape: pl.BlockSpec(shape, lambda i: (0, 0))
    w = full((512, 512))
    b = full((1, 512))
    return pl.pallas_call(
        _tail_body,
        grid=(GRID,),
        in_specs=[row, row, row, row, row, row, row, row,
                  row, row,
                  full((400, 512)), w, b, w, w, w, b, w, b,
                  full((512, 128)), full((1, 128))],
        out_specs=pl.BlockSpec((NB, 1), lambda i: (i, 0)),
        out_shape=jax.ShapeDtypeStruct((N, 1), jnp.float32),
    )(*ts, *hs, ca, cb, zc0, Wl2T, bl2, Wr2T, Wc0xT, Wc1T, bc1, Wc2T, bc2,
      Wc3p, bc3)


def kernel(x, edge_index, z, Wz1, bz1, Wz2, bz2, Wl1, bl1, Wr1, Wl2, bl2,
           Wr2, Wc0, bc0, Wc1, bc1, Wc2, bc2, Wc3, bc3):
    f32 = jnp.float32
    src = edge_index[0].reshape(NSUB, NBLK, 1, BE)
    dst = edge_index[1].reshape(NSUB, NBLK, 1, BE)
    dstc = edge_index[1].reshape(2, NSUB, NCBLK, 1, BC)
    x0 = x[:, :128]
    x1 = x[:, 128:]
    zeros128 = jnp.zeros((80, 128), f32)
    ones128 = jnp.ones((BC, 128), f32)

    ca, cb = _counts(dstc, zeros128, ones128)
    s0, s1 = _segsum2(x0, x1, src, dst, zeros128)

    zc0 = _zhead(z, Wz1.T, bz1.reshape(1, 256), Wz2.T,
                 bz2.reshape(1, 256), Wc0[:, 512:].T, bc0.reshape(1, 512))

    hs = _conv1(s0, s1, ca, cb, x0, x1, Wl1.T, bl1.reshape(1, 512), Wr1.T)

    ts = _segsum4(hs[0], hs[1], hs[2], hs[3], src, dst, zeros128)

    Wc3p = jnp.pad(Wc3.T, ((0, 0), (0, 127)))
    bc3p = jnp.pad(bc3.reshape(1, 1), ((0, 0), (0, 127)))
    out = _tail(ts, hs, ca, cb, zc0, Wl2.T, bl2.reshape(1, 512), Wr2.T,
                Wc0[:, :512].T, Wc1.T, bc1.reshape(1, 512), Wc2.T,
                bc2.reshape(1, 512), Wc3p, bc3p)
    return out


# speedup vs baseline: 5.7856x; 1.2765x over previous
import functools

import jax
import jax.numpy as jnp
from jax import lax
from jax.experimental import pallas as pl
from jax.experimental.pallas import tpu as pltpu
from jax.experimental.pallas import tpu_sc as plsc

N = 10000
E = 160000
NSUB = 16
ROWS_A = 624
ROWS_B = N - ROWS_A * (NSUB - 1)
EDGES_PER_TILE = E // NSUB
BE = 80
NBLK = EDGES_PER_TILE // BE
NB = 2000
GRID = N // NB


def _elu(a):
    return jnp.where(a > 0, a, jnp.exp(a) - 1.0)


def _split_chunks(sid, do):
    @pl.when(sid < NSUB - 1)
    def _():
        for off, ln in [(k * 80, 80) for k in range(7)] + [(560, 64)]:
            do(off, ln)

    @pl.when(sid == NSUB - 1)
    def _():
        for off, ln in [(k * 80, 80) for k in range(8)]:
            do(off, ln)


NBUF = 4
OUTER = NBLK // NBUF


def _make_segsum(num_chunks, chunks_per_core):
    mesh = plsc.VectorSubcoreMesh(core_axis_name="c", subcore_axis_name="s", num_cores=2, num_subcores=16)
    out_type = [jax.ShapeDtypeStruct((N, 128), jnp.float32)
                for _ in range(num_chunks)]
    scratch = [
        pltpu.VMEM_SHARED((N, 128), jnp.float32),
    ] + [pltpu.VMEM((BE, 128), jnp.float32)] * NBUF \
      + [pltpu.VMEM((2, BE), jnp.int32)] * NBUF \
      + [pltpu.SemaphoreType.DMA] * (2 * NBUF)

    @functools.partial(pl.kernel, mesh=mesh, out_type=tuple(out_type),
                       scratch_types=tuple(scratch))
    def seg(*refs):
        tables = refs[:num_chunks]
        sdr, zeros128 = refs[num_chunks:num_chunks + 2]
        outs = refs[num_chunks + 2:2 * num_chunks + 2]
        acc = refs[2 * num_chunks + 2]
        rest = refs[2 * num_chunks + 3:]
        ring = rest[:NBUF]
        sdv = rest[NBUF:2 * NBUF]
        sems = rest[2 * NBUF:3 * NBUF]
        ssems = rest[3 * NBUF:]

        cid = lax.axis_index("c")
        sid = lax.axis_index("s")
        row0 = sid * ROWS_A

        for f in range(num_chunks):
            @pl.when(cid == f // chunks_per_core)
            def _(f=f):
                pltpu.sync_copy(zeros128, ring[0])

                def zinit(off, ln):
                    pltpu.sync_copy(ring[0].at[pl.ds(0, ln)],
                                    acc.at[pl.ds(row0 + off, ln)])

                _split_chunks(sid, zinit)
                plsc.subcore_barrier()

                for b in range(NBUF):
                    pltpu.sync_copy(sdr.at[sid, b], sdv[b])
                    pltpu.async_copy(tables[f].at[sdv[b].at[0]],
                                     ring[b], sems[b])

                def outer(g, carry, f=f):
                    for b in range(NBUF):
                        pltpu.make_async_copy(tables[f].at[sdv[b].at[0]],
                                              ring[b], sems[b]).wait()
                        pltpu.async_copy(ring[b], acc.at[sdv[b].at[1]],
                                        ssems[b], add=True)
                    for b in range(NBUF):
                        j = g * NBUF + b
                        pltpu.make_async_copy(ring[b], acc.at[sdv[b].at[1]],
                                              ssems[b]).wait()

                        @pl.when(j + NBUF < NBLK - 1)
                        def _(b=b, j=j):
                            pltpu.sync_copy(sdr.at[sid, j + NBUF], sdv[b])
                            pltpu.async_copy(tables[f].at[sdv[b].at[0]],
                                             ring[b], sems[b])
                    return carry

                lax.fori_loop(0, OUTER, outer, 0)
                for j in range(OUTER * NBUF, NBLK):
                    pltpu.sync_copy(sdr.at[sid, j], sdv[0])
                    pltpu.async_copy(tables[f].at[sdv[0].at[0]],
                                     ring[0], sems[0]).wait()
                    pltpu.sync_copy(ring[0], acc.at[sdv[0].at[1]], add=True)
                plsc.subcore_barrier()

                def wout(off, ln, f=f):
                    pltpu.sync_copy(acc.at[pl.ds(row0 + off, ln)],
                                    ring[0].at[pl.ds(0, ln)])
                    pltpu.sync_copy(ring[0].at[pl.ds(0, ln)],
                                    outs[f].at[pl.ds(row0 + off, ln)])

                _split_chunks(sid, wout)

    return seg


_segsum2 = _make_segsum(2, 1)
_segsum4 = _make_segsum(4, 2)

BC = 40
EDGES_PER_CTILE = E // 32
NCBLK = EDGES_PER_CTILE // BC

_counts_mesh = plsc.VectorSubcoreMesh(core_axis_name="c", subcore_axis_name="s", num_cores=2, num_subcores=16)


@functools.partial(
    pl.kernel, mesh=_counts_mesh,
    out_type=(jax.ShapeDtypeStruct((N, 128), jnp.float32),
              jax.ShapeDtypeStruct((N, 128), jnp.float32)),
    scratch_types=(
        pltpu.VMEM_SHARED((N, 128), jnp.float32),
        pltpu.VMEM((NCBLK, 1, BC), jnp.int32),
        pltpu.VMEM((BC, 128), jnp.float32),
        pltpu.VMEM((80, 128), jnp.float32),
    ))
def _counts(dstr, zeros128, ones128, out_a, out_b, cacc, dst2d, ones_v, cbuf):
    cid = lax.axis_index("c")
    sid = lax.axis_index("s")
    row0 = sid * ROWS_A
    pltpu.sync_copy(ones128, ones_v)
    pltpu.sync_copy(zeros128, cbuf)
    pltpu.sync_copy(dstr.at[cid, sid], dst2d)

    def zinit(off, ln):
        pltpu.sync_copy(cbuf.at[pl.ds(0, ln)], cacc.at[pl.ds(row0 + off, ln)])

    _split_chunks(sid, zinit)
    plsc.subcore_barrier()

    def body(blk, carry):
        pltpu.sync_copy(ones_v, cacc.at[dst2d.at[blk, 0]], add=True)
        return carry

    lax.fori_loop(0, NCBLK, body, 0)
    plsc.subcore_barrier()

    for core, out in ((0, out_a), (1, out_b)):
        @pl.when(cid == core)
        def _(out=out):
            def cout(off, ln):
                pltpu.sync_copy(cacc.at[pl.ds(row0 + off, ln)],
                                cbuf.at[pl.ds(0, ln)])
                pltpu.sync_copy(cbuf.at[pl.ds(0, ln)],
                                out.at[pl.ds(row0 + off, ln)])

            _split_chunks(sid, cout)


def _dot(a, b):
    return jnp.dot(a, b, preferred_element_type=jnp.float32)


def _zhead_body(z_ref, wz1_ref, bz1_ref, wz2_ref, bz2_ref, wcz_ref, bc0_ref,
                out_ref):
    t = _elu(_dot(z_ref[...], wz1_ref[...]) + bz1_ref[...])
    t = _elu(_dot(t, wz2_ref[...]) + bz2_ref[...])
    out_ref[...] = _dot(t, wcz_ref[...]) + bc0_ref[...]


def _zhead(z, Wz1T, bz1, Wz2T, bz2, WczT, bc0):
    return pl.pallas_call(
        _zhead_body,
        out_shape=jax.ShapeDtypeStruct((400, 512), jnp.float32),
    )(z, Wz1T, bz1, Wz2T, bz2, WczT, bc0)


def _conv1_body(s0_ref, s1_ref, ca_ref, cb_ref, x0_ref, x1_ref, wl_ref,
                bl_ref, wr_ref, h0_ref, h1_ref, h2_ref, h3_ref):
    inv = 1.0 / jnp.clip(ca_ref[:, :1] + cb_ref[:, :1], 1.0, None)
    a = _dot(s0_ref[...] * inv, wl_ref[:128])
    a += _dot(s1_ref[...] * inv, wl_ref[128:])
    a += _dot(x0_ref[...], wr_ref[:128])
    a += _dot(x1_ref[...], wr_ref[128:])
    h = jnp.maximum(a + bl_ref[...], 0.0)
    h0_ref[...] = h[:, 0:128]
    h1_ref[...] = h[:, 128:256]
    h2_ref[...] = h[:, 256:384]
    h3_ref[...] = h[:, 384:512]


def _conv1(s0, s1, ca, cb, x0, x1, Wl1T, bl1, Wr1T):
    row = pl.BlockSpec((NB, 128), lambda i: (i, 0))
    full = lambda shape: pl.BlockSpec(shape, lambda i: (0, 0))
    cspec = pl.BlockSpec((NB, 128), lambda i: (i, 0))
    return pl.pallas_call(
        _conv1_body,
        grid=(GRID,),
        in_specs=[row, row, cspec, cspec,
                  row, row, full((256, 512)), full((1, 512)), full((256, 512))],
        out_specs=[row, row, row, row],
        out_shape=[jax.ShapeDtypeStruct((N, 128), jnp.float32)] * 4,
    )(s0, s1, ca, cb, x0, x1, Wl1T, bl1, Wr1T)


def _tail_body(t0_ref, t1_ref, t2_ref, t3_ref, h0_ref, h1_ref, h2_ref, h3_ref,
               ca_ref, cb_ref, zc_ref, wl_ref, bl_ref, wr_ref, wc0_ref,
               wc1_ref, bc1_ref, wc2_ref, bc2_ref, wc3_ref, bc3_ref, out_ref):
    inv = 1.0 / jnp.clip(ca_ref[:, :1] + cb_ref[:, :1], 1.0, None)
    t_refs = (t0_ref, t1_ref, t2_ref, t3_ref)
    h_refs = (h0_ref, h1_ref, h2_ref, h3_ref)
    a = bl_ref[...] + jnp.zeros((NB, 512), jnp.float32)
    for f in range(4):
        a += _dot(t_refs[f][...] * inv, wl_ref[pl.ds(128 * f, 128)])
        a += _dot(h_refs[f][...], wr_ref[pl.ds(128 * f, 128)])
    c = _dot(a, wc0_ref[...])
    c = (c.reshape(NB // 400, 400, 512) + zc_ref[...][None]).reshape(NB, 512)
    c = _elu(c)
    c = _elu(_dot(c, wc1_ref[...]) + bc1_ref[...])
    c = _elu(_dot(c, wc2_ref[...]) + bc2_ref[...])
    o = jax.nn.sigmoid(_elu(_dot(c, wc3_ref[...]) + bc3_ref[...]))
    out_ref[...] = o[:, 0:1]


def _tail(ts, hs, ca, cb, zc0, Wl2T, bl2, Wr2T, Wc0xT, Wc1T, bc1, Wc2T, bc2,
          Wc3p, bc3):
    row = pl.BlockSpec((NB, 128), lambda i: (i, 0))
    full = lambda shape: pl.BlockSpec(shape, lambda i: (0, 0))
    w = full((512, 512))
    b = full((1, 512))
    return pl.pallas_call(
        _tail_body,
        grid=(GRID,),
        in_specs=[row, row, row, row, row, row, row, row,
                  row, row,
                  full((400, 512)), w, b, w, w, w, b, w, b,
                  full((512, 128)), full((1, 128))],
        out_specs=pl.BlockSpec((NB, 1), lambda i: (i, 0)),
        out_shape=jax.ShapeDtypeStruct((N, 1), jnp.float32),
    )(*ts, *hs, ca, cb, zc0, Wl2T, bl2, Wr2T, Wc0xT, Wc1T, bc1, Wc2T, bc2,
      Wc3p, bc3)


def kernel(x, edge_index, z, Wz1, bz1, Wz2, bz2, Wl1, bl1, Wr1, Wl2, bl2,
           Wr2, Wc0, bc0, Wc1, bc1, Wc2, bc2, Wc3, bc3):
    f32 = jnp.float32
    sd = jnp.stack([edge_index[0].reshape(NSUB, NBLK, BE),
                    edge_index[1].reshape(NSUB, NBLK, BE)], axis=2)
    dstc = edge_index[1].reshape(2, NSUB, NCBLK, 1, BC)
    x0 = x[:, :128]
    x1 = x[:, 128:]
    zeros128 = jnp.zeros((80, 128), f32)
    ones128 = jnp.ones((BC, 128), f32)

    ca, cb = _counts(dstc, zeros128, ones128)
    s0, s1 = _segsum2(x0, x1, sd, zeros128)

    zc0 = _zhead(z, Wz1.T, bz1.reshape(1, 256), Wz2.T,
                 bz2.reshape(1, 256), Wc0[:, 512:].T, bc0.reshape(1, 512))

    hs = _conv1(s0, s1, ca, cb, x0, x1, Wl1.T, bl1.reshape(1, 512), Wr1.T)

    ts = _segsum4(hs[0], hs[1], hs[2], hs[3], sd, zeros128)

    Wc3p = jnp.pad(Wc3.T, ((0, 0), (0, 127)))
    bc3p = jnp.pad(bc3.reshape(1, 1), ((0, 0), (0, 127)))
    out = _tail(ts, hs, ca, cb, zc0, Wl2.T, bl2.reshape(1, 512), Wr2.T,
                Wc0[:, :512].T, Wc1.T, bc1.reshape(1, 512), Wc2.T,
                bc2.reshape(1, 512), Wc3p, bc3p)
    return out


# speedup vs baseline: 6.0996x; 1.0543x over previous
import functools

import jax
import jax.numpy as jnp
from jax import lax
from jax.experimental import pallas as pl
from jax.experimental.pallas import tpu as pltpu
from jax.experimental.pallas import tpu_sc as plsc

N = 10000
E = 160000
NSUB = 16
ROWS_A = 624
ROWS_B = N - ROWS_A * (NSUB - 1)
EDGES_PER_TILE = E // NSUB
BE = 80
NBLK = EDGES_PER_TILE // BE
NB = 2000
GRID = N // NB


def _elu(a):
    return jnp.where(a > 0, a, jnp.exp(a) - 1.0)


def _split_chunks(sid, do):
    @pl.when(sid < NSUB - 1)
    def _():
        for off, ln in [(k * 80, 80) for k in range(7)] + [(560, 64)]:
            do(off, ln)

    @pl.when(sid == NSUB - 1)
    def _():
        for off, ln in [(k * 80, 80) for k in range(8)]:
            do(off, ln)


BEW = 128
NW = E // BEW
NWT = NW // NSUB
NBUF = 3
OUTER = NWT // NBUF


def _make_segsum(num_chunks, chunks_per_core):
    mesh = plsc.VectorSubcoreMesh(core_axis_name="c", subcore_axis_name="s", num_cores=2, num_subcores=16)
    out_type = [jax.ShapeDtypeStruct((N, 128), jnp.float32)
                for _ in range(num_chunks)]
    scratch = [
        pltpu.VMEM_SHARED((N, 128), jnp.float32),
    ] + [pltpu.VMEM((BEW, 128), jnp.float32)] * NBUF \
      + [pltpu.VMEM((2, BEW), jnp.int32)] * NBUF \
      + [pltpu.SemaphoreType.DMA] * (2 * NBUF)

    @functools.partial(pl.kernel, mesh=mesh, out_type=tuple(out_type),
                       scratch_types=tuple(scratch))
    def seg(*refs):
        tables = refs[:num_chunks]
        sdr, zeros128 = refs[num_chunks:num_chunks + 2]
        outs = refs[num_chunks + 2:2 * num_chunks + 2]
        acc = refs[2 * num_chunks + 2]
        rest = refs[2 * num_chunks + 3:]
        ring = rest[:NBUF]
        sdv = rest[NBUF:2 * NBUF]
        sems = rest[2 * NBUF:3 * NBUF]
        ssems = rest[3 * NBUF:]

        cid = lax.axis_index("c")
        sid = lax.axis_index("s")
        row0 = sid * ROWS_A

        for f in range(num_chunks):
            @pl.when(cid == f // chunks_per_core)
            def _(f=f):
                pltpu.sync_copy(zeros128, ring[0].at[pl.ds(0, 80)])

                def zinit(off, ln):
                    pltpu.sync_copy(ring[0].at[pl.ds(0, ln)],
                                    acc.at[pl.ds(row0 + off, ln)])

                _split_chunks(sid, zinit)
                plsc.subcore_barrier()

                w0 = sid * NWT
                for b in range(NBUF):
                    pltpu.sync_copy(sdr.at[w0 + b], sdv[b])
                    pltpu.async_copy(tables[f].at[sdv[b].at[0]],
                                     ring[b], sems[b])

                def outer(g, carry, f=f):
                    for b in range(NBUF):
                        pltpu.make_async_copy(tables[f].at[sdv[b].at[0]],
                                              ring[b], sems[b]).wait()
                        pltpu.async_copy(ring[b], acc.at[sdv[b].at[1]],
                                        ssems[b], add=True)
                    for b in range(NBUF):
                        j = g * NBUF + b
                        pltpu.make_async_copy(ring[b], acc.at[sdv[b].at[1]],
                                              ssems[b]).wait()

                        @pl.when(j + NBUF < NWT)
                        def _(b=b, j=j):
                            pltpu.sync_copy(sdr.at[w0 + j + NBUF], sdv[b])
                            pltpu.async_copy(tables[f].at[sdv[b].at[0]],
                                             ring[b], sems[b])
                    return carry

                lax.fori_loop(0, OUTER, outer, 0)

                @pl.when(sid < NW - NWT * NSUB)
                def _(f=f):
                    pltpu.sync_copy(sdr.at[NWT * NSUB + sid], sdv[0])
                    pltpu.async_copy(tables[f].at[sdv[0].at[0]],
                                     ring[0], sems[0]).wait()
                    pltpu.sync_copy(ring[0], acc.at[sdv[0].at[1]], add=True)
                plsc.subcore_barrier()

                def wout(off, ln, f=f):
                    pltpu.sync_copy(acc.at[pl.ds(row0 + off, ln)],
                                    ring[0].at[pl.ds(0, ln)])
                    pltpu.sync_copy(ring[0].at[pl.ds(0, ln)],
                                    outs[f].at[pl.ds(row0 + off, ln)])

                _split_chunks(sid, wout)

    return seg


_segsum2 = _make_segsum(2, 1)
_segsum4 = _make_segsum(4, 2)

BC = 40
EDGES_PER_CTILE = E // 32
NCBLK = EDGES_PER_CTILE // BC

_counts_mesh = plsc.VectorSubcoreMesh(core_axis_name="c", subcore_axis_name="s", num_cores=2, num_subcores=16)


@functools.partial(
    pl.kernel, mesh=_counts_mesh,
    out_type=(jax.ShapeDtypeStruct((N, 128), jnp.float32),
              jax.ShapeDtypeStruct((N, 128), jnp.float32)),
    scratch_types=(
        pltpu.VMEM_SHARED((N, 128), jnp.float32),
        pltpu.VMEM((NCBLK, 1, BC), jnp.int32),
        pltpu.VMEM((BC, 128), jnp.float32),
        pltpu.VMEM((80, 128), jnp.float32),
    ))
def _counts(dstr, zeros128, ones128, out_a, out_b, cacc, dst2d, ones_v, cbuf):
    cid = lax.axis_index("c")
    sid = lax.axis_index("s")
    row0 = sid * ROWS_A
    pltpu.sync_copy(ones128, ones_v)
    pltpu.sync_copy(zeros128, cbuf)
    pltpu.sync_copy(dstr.at[cid, sid], dst2d)

    def zinit(off, ln):
        pltpu.sync_copy(cbuf.at[pl.ds(0, ln)], cacc.at[pl.ds(row0 + off, ln)])

    _split_chunks(sid, zinit)
    plsc.subcore_barrier()

    def body(blk, carry):
        pltpu.sync_copy(ones_v, cacc.at[dst2d.at[blk, 0]], add=True)
        return carry

    lax.fori_loop(0, NCBLK, body, 0)
    plsc.subcore_barrier()

    for core, out in ((0, out_a), (1, out_b)):
        @pl.when(cid == core)
        def _(out=out):
            def cout(off, ln):
                pltpu.sync_copy(cacc.at[pl.ds(row0 + off, ln)],
                                cbuf.at[pl.ds(0, ln)])
                pltpu.sync_copy(cbuf.at[pl.ds(0, ln)],
                                out.at[pl.ds(row0 + off, ln)])

            _split_chunks(sid, cout)


def _dot(a, b):
    return jnp.dot(a, b, preferred_element_type=jnp.float32)


def _zhead_body(z_ref, wz1_ref, bz1_ref, wz2_ref, bz2_ref, wcz_ref, bc0_ref,
                out_ref):
    t = _elu(_dot(z_ref[...], wz1_ref[...]) + bz1_ref[...])
    t = _elu(_dot(t, wz2_ref[...]) + bz2_ref[...])
    out_ref[...] = _dot(t, wcz_ref[...]) + bc0_ref[...]


def _zhead(z, Wz1T, bz1, Wz2T, bz2, WczT, bc0):
    return pl.pallas_call(
        _zhead_body,
        out_shape=jax.ShapeDtypeStruct((400, 512), jnp.float32),
    )(z, Wz1T, bz1, Wz2T, bz2, WczT, bc0)


def _conv1_body(s0_ref, s1_ref, ca_ref, cb_ref, x0_ref, x1_ref, wl_ref,
                bl_ref, wr_ref, h0_ref, h1_ref, h2_ref, h3_ref):
    inv = 1.0 / jnp.clip(ca_ref[:, :1] + cb_ref[:, :1], 1.0, None)
    a = _dot(s0_ref[...] * inv, wl_ref[:128])
    a += _dot(s1_ref[...] * inv, wl_ref[128:])
    a += _dot(x0_ref[...], wr_ref[:128])
    a += _dot(x1_ref[...], wr_ref[128:])
    h = jnp.maximum(a + bl_ref[...], 0.0)
    h0_ref[...] = h[:, 0:128]
    h1_ref[...] = h[:, 128:256]
    h2_ref[...] = h[:, 256:384]
    h3_ref[...] = h[:, 384:512]


def _conv1(s0, s1, ca, cb, x0, x1, Wl1T, bl1, Wr1T):
    row = pl.BlockSpec((NB, 128), lambda i: (i, 0))
    full = lambda shape: pl.BlockSpec(shape, lambda i: (0, 0))
    cspec = pl.BlockSpec((NB, 128), lambda i: (i, 0))
    return pl.pallas_call(
        _conv1_body,
        grid=(GRID,),
        in_specs=[row, row, cspec, cspec,
                  row, row, full((256, 512)), full((1, 512)), full((256, 512))],
        out_specs=[row, row, row, row],
        out_shape=[jax.ShapeDtypeStruct((N, 128), jnp.float32)] * 4,
    )(s0, s1, ca, cb, x0, x1, Wl1T, bl1, Wr1T)


def _tail_body(t0_ref, t1_ref, t2_ref, t3_ref, h0_ref, h1_ref, h2_ref, h3_ref,
               ca_ref, cb_ref, zc_ref, wl_ref, bl_ref, wr_ref, wc0_ref,
               wc1_ref, bc1_ref, wc2_ref, bc2_ref, wc3_ref, bc3_ref, out_ref):
    inv = 1.0 / jnp.clip(ca_ref[:, :1] + cb_ref[:, :1], 1.0, None)
    t_refs = (t0_ref, t1_ref, t2_ref, t3_ref)
    h_refs = (h0_ref, h1_ref, h2_ref, h3_ref)
    a = bl_ref[...] + jnp.zeros((NB, 512), jnp.float32)
    for f in range(4):
        a += _dot(t_refs[f][...] * inv, wl_ref[pl.ds(128 * f, 128)])
        a += _dot(h_refs[f][...], wr_ref[pl.ds(128 * f, 128)])
    c = _dot(a, wc0_ref[...])
    c = (c.reshape(NB // 400, 400, 512) + zc_ref[...][None]).reshape(NB, 512)
    c = _elu(c)
    c = _elu(_dot(c, wc1_ref[...]) + bc1_ref[...])
    c = _elu(_dot(c, wc2_ref[...]) + bc2_ref[...])
    o = jax.nn.sigmoid(_elu(_dot(c, wc3_ref[...]) + bc3_ref[...]))
    out_ref[...] = o[:, 0:1]


def _tail(ts, hs, ca, cb, zc0, Wl2T, bl2, Wr2T, Wc0xT, Wc1T, bc1, Wc2T, bc2,
          Wc3p, bc3):
    row = pl.BlockSpec((NB, 128), lambda i: (i, 0))
    full = lambda shape: pl.BlockSpec(shape, lambda i: (0, 0))
    w = full((512, 512))
    b = full((1, 512))
    return pl.pallas_call(
        _tail_body,
        grid=(GRID,),
        in_specs=[row, row, row, row, row, row, row, row,
                  row, row,
                  full((400, 512)), w, b, w, w, w, b, w, b,
                  full((512, 128)), full((1, 128))],
        out_specs=pl.BlockSpec((NB, 1), lambda i: (i, 0)),
        out_shape=jax.ShapeDtypeStruct((N, 1), jnp.float32),
    )(*ts, *hs, ca, cb, zc0, Wl2T, bl2, Wr2T, Wc0xT, Wc1T, bc1, Wc2T, bc2,
      Wc3p, bc3)


def kernel(x, edge_index, z, Wz1, bz1, Wz2, bz2, Wl1, bl1, Wr1, Wl2, bl2,
           Wr2, Wc0, bc0, Wc1, bc1, Wc2, bc2, Wc3, bc3):
    f32 = jnp.float32
    sd = jnp.stack([edge_index[0].reshape(NW, BEW),
                    edge_index[1].reshape(NW, BEW)], axis=1)
    dstc = edge_index[1].reshape(2, NSUB, NCBLK, 1, BC)
    x0 = x[:, :128]
    x1 = x[:, 128:]
    zeros128 = jnp.zeros((80, 128), f32)
    ones128 = jnp.ones((BC, 128), f32)

    ca, cb = _counts(dstc, zeros128, ones128)
    s0, s1 = _segsum2(x0, x1, sd, zeros128)

    zc0 = _zhead(z, Wz1.T, bz1.reshape(1, 256), Wz2.T,
                 bz2.reshape(1, 256), Wc0[:, 512:].T, bc0.reshape(1, 512))

    hs = _conv1(s0, s1, ca, cb, x0, x1, Wl1.T, bl1.reshape(1, 512), Wr1.T)

    ts = _segsum4(hs[0], hs[1], hs[2], hs[3], sd, zeros128)

    Wc3p = jnp.pad(Wc3.T, ((0, 0), (0, 127)))
    bc3p = jnp.pad(bc3.reshape(1, 1), ((0, 0), (0, 127)))
    out = _tail(ts, hs, ca, cb, zc0, Wl2.T, bl2.reshape(1, 512), Wr2.T,
                Wc0[:, :512].T, Wc1.T, bc1.reshape(1, 512), Wc2.T,
                bc2.reshape(1, 512), Wc3p, bc3p)
    return out


# speedup vs baseline: 6.2691x; 1.0278x over previous
import functools

import jax
import jax.numpy as jnp
from jax import lax
from jax.experimental import pallas as pl
from jax.experimental.pallas import tpu as pltpu
from jax.experimental.pallas import tpu_sc as plsc

N = 10000
E = 160000
NSUB = 16
ROWS_A = 624
ROWS_B = N - ROWS_A * (NSUB - 1)
EDGES_PER_TILE = E // NSUB
BE = 80
NBLK = EDGES_PER_TILE // BE
NB = 2000
GRID = N // NB


def _elu(a):
    return jnp.where(a > 0, a, jnp.exp(a) - 1.0)


def _split_chunks(sid, do):
    @pl.when(sid < NSUB - 1)
    def _():
        for off, ln in [(k * 80, 80) for k in range(7)] + [(560, 64)]:
            do(off, ln)

    @pl.when(sid == NSUB - 1)
    def _():
        for off, ln in [(k * 80, 80) for k in range(8)]:
            do(off, ln)


BEW = 128
NW = E // BEW
NWT = NW // NSUB
NBUF = 3
OUTER = NWT // NBUF


NWC = (NW // 2) // NSUB


def _make_segsum(num_chunks, chunks_per_core, with_counts=False):
    mesh = plsc.VectorSubcoreMesh(core_axis_name="c", subcore_axis_name="s", num_cores=2, num_subcores=16)
    n_out = num_chunks + (2 if with_counts else 0)
    out_type = [jax.ShapeDtypeStruct((N, 128), jnp.float32)
                for _ in range(n_out)]
    scratch = [
        pltpu.VMEM_SHARED((N, 128), jnp.float32),
    ] + [pltpu.VMEM((BEW, 128), jnp.float32)] * NBUF \
      + [pltpu.VMEM((2, BEW), jnp.int32)] * NBUF \
      + [pltpu.SemaphoreType.DMA] * (2 * NBUF)

    @functools.partial(pl.kernel, mesh=mesh, out_type=tuple(out_type),
                       scratch_types=tuple(scratch))
    def seg(*refs):
        tables = refs[:num_chunks]
        p = num_chunks
        sdr, zeros128 = refs[p:p + 2]
        p += 2
        if with_counts:
            ones128 = refs[p]
            p += 1
        outs = refs[p:p + num_chunks]
        p += num_chunks
        if with_counts:
            couts = refs[p:p + 2]
            p += 2
        acc = refs[p]
        rest = refs[p + 1:]
        ring = rest[:NBUF]
        sdv = rest[NBUF:2 * NBUF]
        sems = rest[2 * NBUF:3 * NBUF]
        ssems = rest[3 * NBUF:]

        cid = lax.axis_index("c")
        sid = lax.axis_index("s")
        row0 = sid * ROWS_A

        for f in range(num_chunks):
            @pl.when(cid == f // chunks_per_core)
            def _(f=f):
                pltpu.sync_copy(zeros128, ring[0].at[pl.ds(0, 80)])

                def zinit(off, ln):
                    pltpu.sync_copy(ring[0].at[pl.ds(0, ln)],
                                    acc.at[pl.ds(row0 + off, ln)])

                _split_chunks(sid, zinit)
                plsc.subcore_barrier()

                w0 = sid * NWT
                for b in range(NBUF):
                    pltpu.sync_copy(sdr.at[w0 + b], sdv[b])
                    pltpu.async_copy(tables[f].at[sdv[b].at[0]],
                                     ring[b], sems[b])

                def outer(g, carry, f=f):
                    for b in range(NBUF):
                        pltpu.make_async_copy(tables[f].at[sdv[b].at[0]],
                                              ring[b], sems[b]).wait()
                        pltpu.async_copy(ring[b], acc.at[sdv[b].at[1]],
                                        ssems[b], add=True)
                    for b in range(NBUF):
                        j = g * NBUF + b
                        pltpu.make_async_copy(ring[b], acc.at[sdv[b].at[1]],
                                              ssems[b]).wait()

                        @pl.when(j + NBUF < NWT)
                        def _(b=b, j=j):
                            pltpu.sync_copy(sdr.at[w0 + j + NBUF], sdv[b])
                            pltpu.async_copy(tables[f].at[sdv[b].at[0]],
                                             ring[b], sems[b])
                    return carry

                lax.fori_loop(0, OUTER, outer, 0)

                @pl.when(sid < NW - NWT * NSUB)
                def _(f=f):
                    pltpu.sync_copy(sdr.at[NWT * NSUB + sid], sdv[0])
                    pltpu.async_copy(tables[f].at[sdv[0].at[0]],
                                     ring[0], sems[0]).wait()
                    pltpu.sync_copy(ring[0], acc.at[sdv[0].at[1]], add=True)
                plsc.subcore_barrier()

                def wout(off, ln, f=f):
                    pltpu.sync_copy(acc.at[pl.ds(row0 + off, ln)],
                                    ring[0].at[pl.ds(0, ln)])
                    pltpu.sync_copy(ring[0].at[pl.ds(0, ln)],
                                    outs[f].at[pl.ds(row0 + off, ln)])

                _split_chunks(sid, wout)

        if with_counts:
            pltpu.sync_copy(zeros128, ring[0].at[pl.ds(0, 80)])

            def czinit(off, ln):
                pltpu.sync_copy(ring[0].at[pl.ds(0, ln)],
                                acc.at[pl.ds(row0 + off, ln)])

            _split_chunks(sid, czinit)
            plsc.subcore_barrier()
            pltpu.sync_copy(ones128, ring[1])
            cw0 = cid * (NW // 2) + sid * NWC
            for b in range(NBUF):
                pltpu.sync_copy(sdr.at[cw0 + b], sdv[b])
                pltpu.async_copy(ring[1], acc.at[sdv[b].at[1]],
                                 ssems[b], add=True)

            def couter(g, carry):
                for b in range(NBUF):
                    j = g * NBUF + b
                    pltpu.make_async_copy(ring[1], acc.at[sdv[b].at[1]],
                                          ssems[b]).wait()

                    @pl.when(j + NBUF < NWC)
                    def _(b=b, j=j):
                        pltpu.sync_copy(sdr.at[cw0 + j + NBUF], sdv[b])
                        pltpu.async_copy(ring[1], acc.at[sdv[b].at[1]],
                                         ssems[b], add=True)
                return carry

            lax.fori_loop(0, NWC // NBUF, couter, 0)

            @pl.when(sid == 0)
            def _():
                pltpu.sync_copy(sdr.at[cid * (NW // 2) + NSUB * NWC], sdv[0])
                pltpu.sync_copy(ring[1], acc.at[sdv[0].at[1]], add=True)
            plsc.subcore_barrier()

            for core in (0, 1):
                @pl.when(cid == core)
                def _(core=core):
                    def cwout(off, ln, core=core):
                        pltpu.sync_copy(acc.at[pl.ds(row0 + off, ln)],
                                        ring[0].at[pl.ds(0, ln)])
                        pltpu.sync_copy(ring[0].at[pl.ds(0, ln)],
                                        couts[core].at[pl.ds(row0 + off, ln)])

                    _split_chunks(sid, cwout)

    return seg


_segsum2 = _make_segsum(2, 1, with_counts=True)
_segsum4 = _make_segsum(4, 2)


def _dot(a, b):
    return jnp.dot(a, b, preferred_element_type=jnp.float32)


def _zhead_body(z_ref, wz1_ref, bz1_ref, wz2_ref, bz2_ref, wcz_ref, bc0_ref,
                out_ref):
    t = _elu(_dot(z_ref[...], wz1_ref[...]) + bz1_ref[...])
    t = _elu(_dot(t, wz2_ref[...]) + bz2_ref[...])
    out_ref[...] = _dot(t, wcz_ref[...]) + bc0_ref[...]


def _zhead(z, Wz1T, bz1, Wz2T, bz2, WczT, bc0):
    return pl.pallas_call(
        _zhead_body,
        out_shape=jax.ShapeDtypeStruct((400, 512), jnp.float32),
    )(z, Wz1T, bz1, Wz2T, bz2, WczT, bc0)


def _conv1_body(s0_ref, s1_ref, ca_ref, cb_ref, x0_ref, x1_ref, wl_ref,
                bl_ref, wr_ref, h0_ref, h1_ref, h2_ref, h3_ref):
    inv = 1.0 / jnp.clip(ca_ref[:, :1] + cb_ref[:, :1], 1.0, None)
    a = _dot(s0_ref[...] * inv, wl_ref[:128])
    a += _dot(s1_ref[...] * inv, wl_ref[128:])
    a += _dot(x0_ref[...], wr_ref[:128])
    a += _dot(x1_ref[...], wr_ref[128:])
    h = jnp.maximum(a + bl_ref[...], 0.0)
    h0_ref[...] = h[:, 0:128]
    h1_ref[...] = h[:, 128:256]
    h2_ref[...] = h[:, 256:384]
    h3_ref[...] = h[:, 384:512]


def _conv1(s0, s1, ca, cb, x0, x1, Wl1T, bl1, Wr1T):
    row = pl.BlockSpec((NB, 128), lambda i: (i, 0))
    full = lambda shape: pl.BlockSpec(shape, lambda i: (0, 0))
    cspec = pl.BlockSpec((NB, 128), lambda i: (i, 0))
    return pl.pallas_call(
        _conv1_body,
        grid=(GRID,),
        in_specs=[row, row, cspec, cspec,
                  row, row, full((256, 512)), full((1, 512)), full((256, 512))],
        out_specs=[row, row, row, row],
        out_shape=[jax.ShapeDtypeStruct((N, 128), jnp.float32)] * 4,
    )(s0, s1, ca, cb, x0, x1, Wl1T, bl1, Wr1T)


def _tail_body(t0_ref, t1_ref, t2_ref, t3_ref, h0_ref, h1_ref, h2_ref, h3_ref,
               ca_ref, cb_ref, zc_ref, wl_ref, bl_ref, wr_ref, wc0_ref,
               wc1_ref, bc1_ref, wc2_ref, bc2_ref, wc3_ref, bc3_ref, out_ref):
    inv = 1.0 / jnp.clip(ca_ref[:, :1] + cb_ref[:, :1], 1.0, None)
    t_refs = (t0_ref, t1_ref, t2_ref, t3_ref)
    h_refs = (h0_ref, h1_ref, h2_ref, h3_ref)
    a = bl_ref[...] + jnp.zeros((NB, 512), jnp.float32)
    for f in range(4):
        a += _dot(t_refs[f][...] * inv, wl_ref[pl.ds(128 * f, 128)])
        a += _dot(h_refs[f][...], wr_ref[pl.ds(128 * f, 128)])
    c = _dot(a, wc0_ref[...])
    c = (c.reshape(NB // 400, 400, 512) + zc_ref[...][None]).reshape(NB, 512)
    c = _elu(c)
    c = _elu(_dot(c, wc1_ref[...]) + bc1_ref[...])
    c = _elu(_dot(c, wc2_ref[...]) + bc2_ref[...])
    o = jax.nn.sigmoid(_elu(_dot(c, wc3_ref[...]) + bc3_ref[...]))
    out_ref[...] = o[:, 0:1]


def _tail(ts, hs, ca, cb, zc0, Wl2T, bl2, Wr2T, Wc0xT, Wc1T, bc1, Wc2T, bc2,
          Wc3p, bc3):
    row = pl.BlockSpec((NB, 128), lambda i: (i, 0))
    full = lambda shape: pl.BlockSpec(shape, lambda i: (0, 0))
    w = full((512, 512))
    b = full((1, 512))
    return pl.pallas_call(
        _tail_body,
        grid=(GRID,),
        in_specs=[row, row, row, row, row, row, row, row,
                  row, row,
                  full((400, 512)), w, b, w, w, w, b, w, b,
                  full((512, 128)), full((1, 128))],
        out_specs=pl.BlockSpec((NB, 1), lambda i: (i, 0)),
        out_shape=jax.ShapeDtypeStruct((N, 1), jnp.float32),
    )(*ts, *hs, ca, cb, zc0, Wl2T, bl2, Wr2T, Wc0xT, Wc1T, bc1, Wc2T, bc2,
      Wc3p, bc3)


def kernel(x, edge_index, z, Wz1, bz1, Wz2, bz2, Wl1, bl1, Wr1, Wl2, bl2,
           Wr2, Wc0, bc0, Wc1, bc1, Wc2, bc2, Wc3, bc3):
    f32 = jnp.float32
    sd = jnp.stack([edge_index[0].reshape(NW, BEW),
                    edge_index[1].reshape(NW, BEW)], axis=1)
    x0 = x[:, :128]
    x1 = x[:, 128:]
    zeros128 = jnp.zeros((80, 128), f32)
    ones128 = jnp.ones((BEW, 128), f32)

    s0, s1, ca, cb = _segsum2(x0, x1, sd, zeros128, ones128)

    zc0 = _zhead(z, Wz1.T, bz1.reshape(1, 256), Wz2.T,
                 bz2.reshape(1, 256), Wc0[:, 512:].T, bc0.reshape(1, 512))

    hs = _conv1(s0, s1, ca, cb, x0, x1, Wl1.T, bl1.reshape(1, 512), Wr1.T)

    ts = _segsum4(hs[0], hs[1], hs[2], hs[3], sd, zeros128)

    Wc3p = jnp.pad(Wc3.T, ((0, 0), (0, 127)))
    bc3p = jnp.pad(bc3.reshape(1, 1), ((0, 0), (0, 127)))
    out = _tail(ts, hs, ca, cb, zc0, Wl2.T, bl2.reshape(1, 512), Wr2.T,
                Wc0[:, :512].T, Wc1.T, bc1.reshape(1, 512), Wc2.T,
                bc2.reshape(1, 512), Wc3p, bc3p)
    return out
